# Initial kernel scaffold; baseline (speedup 1.0000x reference)
import jax
import jax.numpy as jnp
from jax import lax
from jax.experimental import pallas as pl
from jax.experimental.pallas import tpu as pltpu
from jax.experimental.pallas import tpu_sc as plsc

B, S, H = 1024, 200, 768
BS = B * S
EPS = 1e-12
L = 16
NV = H // L
NC, NS = 2, 16
NW = NC * NS
K = 50
PBLK = S // K
SEQ_PER_W = B // NW
NBLK = BS // K


def _rsqrt_newton(v):
    iv = lax.bitcast_convert_type(v, jnp.int32)
    y = lax.bitcast_convert_type(jnp.int32(0x5F3759DF) - (iv >> 1), jnp.float32)
    for _ in range(3):
        y = y * (1.5 - 0.5 * v * y * y)
    return y


def _body(idx3d, word_emb, x_emb, y_emb, h_emb, w_emb, tok_emb, pos_emb,
          ln_g, ln_b, out, idx_buf, acc, pos_buf, g_buf, b_buf, sem):
    wid = lax.axis_index("s") * NC + lax.axis_index("c")

    pltpu.sync_copy(ln_g, g_buf)
    pltpu.sync_copy(ln_b, b_buf)

    tables = [word_emb, x_emb, x_emb, y_emb, y_emb, h_emb, w_emb, tok_emb]

    def j_loop(j, _):
        pltpu.sync_copy(pos_emb.at[pl.ds(j * K, K)], pos_buf)

        def b_loop(b, _):
            blk = (wid * SEQ_PER_W + b) * PBLK + j
            pltpu.sync_copy(idx3d.at[blk], idx_buf)
            pltpu.async_copy(tables[0].at[idx_buf.at[0]], acc, sem).wait()
            copies = [
                pltpu.async_copy(tables[t].at[idx_buf.at[t]], acc, sem, add=True)
                for t in range(1, 8)
            ]
            for c in copies:
                c.wait()

            def tok_loop(i, _):
                s = jnp.zeros((L,), jnp.float32)
                s2 = jnp.zeros((L,), jnp.float32)
                for jj in range(NV):
                    sl = pl.ds(jj * L, L)
                    x = acc[i, sl] + pos_buf[i, sl]
                    acc[i, sl] = x
                    s = s + x
                    s2 = s2 + x * x
                mean = jnp.sum(s) * (1.0 / H)
                var = jnp.sum(s2) * (1.0 / H) - mean * mean
                r = _rsqrt_newton(jnp.full((L,), var + EPS, jnp.float32))
                mv = jnp.full((L,), mean, jnp.float32)
                for jj in range(NV):
                    sl = pl.ds(jj * L, L)
                    acc[i, sl] = (acc[i, sl] - mv) * r * g_buf[sl] + b_buf[sl]
                return 0

            lax.fori_loop(0, K, tok_loop, 0)
            pltpu.sync_copy(acc, out.at[pl.ds(blk * K, K)])
            return 0

        lax.fori_loop(0, SEQ_PER_W, b_loop, 0)
        return 0

    lax.fori_loop(0, PBLK, j_loop, 0)


def kernel(input_ids, bbox, token_type_ids, word_emb, pos_emb, x_emb, y_emb,
           h_emb, w_emb, tok_emb, ln_g, ln_b):
    ids = input_ids.reshape(BS).astype(jnp.int32)
    bb = bbox.reshape(BS, 4).astype(jnp.int32)
    x0, y0, x1, y1 = bb[:, 0], bb[:, 1], bb[:, 2], bb[:, 3]
    tt = token_type_ids.reshape(BS).astype(jnp.int32)
    idx_all = jnp.stack([ids, x0, x1, y0, y1, y1 - y0, x1 - x0, tt])
    idx3d = idx_all.reshape(8, NBLK, K).transpose(1, 0, 2)

    fn = pl.kernel(
        _body,
        out_type=jax.ShapeDtypeStruct((BS, H), jnp.float32),
        mesh=plsc.VectorSubcoreMesh(
            core_axis_name="c", subcore_axis_name="s",
            num_cores=NC, num_subcores=NS),
        scratch_types=[
            pltpu.VMEM((8, K), jnp.int32),
            pltpu.VMEM((K, H), jnp.float32),
            pltpu.VMEM((K, H), jnp.float32),
            pltpu.VMEM((H,), jnp.float32),
            pltpu.VMEM((H,), jnp.float32),
            pltpu.SemaphoreType.DMA,
        ],
    )
    out = fn(idx3d, word_emb, x_emb, y_emb, h_emb, w_emb, tok_emb, pos_emb,
             ln_g, ln_b)
    return out.reshape(B, S, H)


# speedup vs baseline: 1.6108x; 1.6108x over previous
import jax
import jax.numpy as jnp
from jax import lax
from jax.experimental import pallas as pl
from jax.experimental.pallas import tpu as pltpu
from jax.experimental.pallas import tpu_sc as plsc

B, S, H = 1024, 200, 768
BS = B * S
EPS = 1e-12
L = 16
NV = H // L
NC, NS = 2, 16
NW = NC * NS
K = 32
NT = 9
NBLK = BS // K
BLK_PER_W = NBLK // NW


def _rsqrt_newton(v):
    iv = lax.bitcast_convert_type(v, jnp.int32)
    y = lax.bitcast_convert_type(jnp.int32(0x5F3759DF) - (iv >> 1), jnp.float32)
    for _ in range(3):
        y = y * (1.5 - 0.5 * v * y * y)
    return y


def _body(idx_flat, word_emb, x_emb, y_emb, h_emb, w_emb, tok_emb, pos_emb,
          ln_g, ln_b, out, idx_buf, acc, buf0, buf1, g_buf, b_buf, sem):
    wid = lax.axis_index("s") * NC + lax.axis_index("c")

    pltpu.sync_copy(ln_g, g_buf)
    pltpu.sync_copy(ln_b, b_buf)

    tables = [word_emb, x_emb, x_emb, y_emb, y_emb, h_emb, w_emb, tok_emb,
              pos_emb]
    bufs = [buf0, buf1]

    def blk_loop(n, _):
        blk = wid * BLK_PER_W + n
        pltpu.sync_copy(idx_flat.at[pl.ds(blk * (NT * K), NT * K)], idx_buf)
        cw = pltpu.async_copy(
            tables[0].at[idx_buf.at[pl.ds(0, K)]], acc, sem)
        copies = {1: pltpu.async_copy(
            tables[1].at[idx_buf.at[pl.ds(K, K)]], bufs[1], sem)}
        cw.wait()
        for t in range(1, NT):
            copies[t].wait()
            if t + 1 < NT:
                copies[t + 1] = pltpu.async_copy(
                    tables[t + 1].at[idx_buf.at[pl.ds((t + 1) * K, K)]],
                    bufs[(t + 1) % 2], sem)
            src = bufs[t % 2]

            def add_loop(i, _, src=src):
                for jj in range(NV):
                    sl = pl.ds(jj * L, L)
                    plsc.addupdate(acc.at[i, sl], src[i, sl])
                return 0

            lax.fori_loop(0, K, add_loop, 0)

        def tok_loop(i, _):
            s = jnp.zeros((L,), jnp.float32)
            s2 = jnp.zeros((L,), jnp.float32)
            for jj in range(NV):
                x = acc[i, pl.ds(jj * L, L)]
                s = s + x
                s2 = s2 + x * x
            mean = jnp.sum(s) * (1.0 / H)
            var = jnp.sum(s2) * (1.0 / H) - mean * mean
            r = _rsqrt_newton(jnp.full((L,), var + EPS, jnp.float32))
            mv = jnp.full((L,), mean, jnp.float32)
            for jj in range(NV):
                sl = pl.ds(jj * L, L)
                acc[i, sl] = (acc[i, sl] - mv) * r * g_buf[sl] + b_buf[sl]
            return 0

        lax.fori_loop(0, K, tok_loop, 0)
        pltpu.sync_copy(acc, out.at[pl.ds(blk * K, K)])
        return 0

    lax.fori_loop(0, BLK_PER_W, blk_loop, 0)


def kernel(input_ids, bbox, token_type_ids, word_emb, pos_emb, x_emb, y_emb,
           h_emb, w_emb, tok_emb, ln_g, ln_b):
    ids = input_ids.reshape(BS).astype(jnp.int32)
    bb = bbox.reshape(BS, 4).astype(jnp.int32)
    x0, y0, x1, y1 = bb[:, 0], bb[:, 1], bb[:, 2], bb[:, 3]
    tt = token_type_ids.reshape(BS).astype(jnp.int32)
    posi = jnp.broadcast_to(jnp.arange(S, dtype=jnp.int32), (B, S)).reshape(BS)
    idx_all = jnp.stack([ids, x0, x1, y0, y1, y1 - y0, x1 - x0, tt, posi])
    idx_flat = idx_all.reshape(NT, NBLK, K).transpose(1, 0, 2).reshape(-1)

    fn = pl.kernel(
        _body,
        out_type=jax.ShapeDtypeStruct((BS, H), jnp.float32),
        mesh=plsc.VectorSubcoreMesh(
            core_axis_name="c", subcore_axis_name="s",
            num_cores=NC, num_subcores=NS),
        scratch_types=[
            pltpu.VMEM((NT * K,), jnp.int32),
            pltpu.VMEM((K, H), jnp.float32),
            pltpu.VMEM((K, H), jnp.float32),
            pltpu.VMEM((K, H), jnp.float32),
            pltpu.VMEM((H,), jnp.float32),
            pltpu.VMEM((H,), jnp.float32),
            pltpu.SemaphoreType.DMA,
        ],
        compiler_params=pltpu.CompilerParams(needs_layout_passes=False),
    )
    out = fn(idx_flat, word_emb, x_emb, y_emb, h_emb, w_emb, tok_emb, pos_emb,
             ln_g, ln_b)
    return out.reshape(B, S, H)
